# Initial kernel scaffold; baseline (speedup 1.0000x reference)
import jax
import jax.numpy as jnp
from jax.experimental import pallas as pl


def kernel(node_features, edge_features, params, from_idx, to_idx):
    raise NotImplementedError("write your pallas kernel here")


# speedup vs baseline: 1.4343x; 1.4343x over previous
import jax
import jax.numpy as jnp
from jax.experimental import pallas as pl
from jax.experimental.pallas import tpu as pltpu

_B = 64
_NN = 30
_NE = 64
_NMAX = 32
_EMAX = 80
_F = 128
_LAYERS = 3
_SITER = 20
_TEMP = 0.1


def _mm(a, b):
    return jax.lax.dot_general(a, b, (((1,), (0,)), ((), ())),
                               preferred_element_type=jnp.float32)


def _mmT(a, b):
    return jax.lax.dot_general(a, b, (((1,), (1,)), ((), ())),
                               preferred_element_type=jnp.float32)


def _gather(oT, x):
    return jax.lax.dot_general(oT, x, (((0,), (0,)), ((), ())),
                               preferred_element_type=jnp.float32)


def _sinkhorn(la):
    for _ in range(_SITER):
        m = jnp.max(la, axis=1, keepdims=True)
        la = la - (m + jnp.log(jnp.sum(jnp.exp(la - m), axis=1, keepdims=True)))
        m = jnp.max(la, axis=0, keepdims=True)
        la = la - (m + jnp.log(jnp.sum(jnp.exp(la - m), axis=0, keepdims=True)))
    return jnp.exp(la)


def _pair_body(nf_ref, ef_ref, fi_ref, ti_ref,
               wen_ref, ben_ref, wee_ref, bee_ref,
               w1_ref, b1_ref, wm2_ref, bm2_ref, wr2_ref, br2_ref,
               wu1_ref, bu1_ref, wu2_ref, bu2_ref,
               wn1_ref, bn1_ref, wn2_ref, bn2_ref,
               we1_ref, be1_ref, we2_ref, be2_ref, wv_ref,
               out_ref):
    nf = nf_ref[0]
    ef = ef_ref[0]
    W1 = w1_ref[...]; B1 = b1_ref[...]
    WM2 = wm2_ref[...]; BM2 = bm2_ref[...]
    WR2 = wr2_ref[...]; BR2 = br2_ref[...]
    WU1 = wu1_ref[...]; BU1 = bu1_ref[...]
    WU2 = wu2_ref[...]; BU2 = bu2_ref[...]

    h = _mm(nf, wen_ref[...]) + ben_ref[...]
    e = _mm(ef, wee_ref[...]) + bee_ref[...]

    niota = jax.lax.broadcasted_iota(jnp.int32, (2 * _NN, 2 * _NE), 0)
    ofT = (niota == fi_ref[0]).astype(jnp.float32)
    otT = (niota == ti_ref[0]).astype(jnp.float32)

    def messages(hcur):
        hf = _gather(ofT, hcur)
        ht = _gather(otT, hcur)
        x = jnp.concatenate([hf, ht, e], axis=1)
        y = jnp.maximum(_mm(x, W1) + B1, 0.0)
        m = _mm(y[:, :2 * _F], WM2) + BM2
        rm = _mm(y[:, 2 * _F:], WR2) + BR2
        return m, rm

    for _ in range(_LAYERS):
        m, rm = messages(h)
        agg = _mm(otT, m) + _mm(ofT, rm)
        u = jnp.concatenate([h, agg], axis=1)
        h = _mm(jnp.maximum(_mm(u, WU1) + BU1, 0.0), WU2) + BU2

    nt = _mm(jnp.maximum(_mm(h, wn1_ref[...]) + bn1_ref[...], 0.0),
             wn2_ref[...]) + bn2_ref[...]
    zn = jnp.zeros((_NMAX - _NN, _F), jnp.float32)
    qn = jnp.concatenate([nt[:_NN], zn], axis=0)
    cn = jnp.concatenate([nt[_NN:], zn], axis=0)
    Pn = _sinkhorn(_mmT(qn, cn) / _TEMP)
    s_node = -jnp.sum(jnp.maximum(qn - _mm(Pn, cn), 0.0))

    m, rm = messages(h)
    em = m + rm
    et = _mm(jnp.maximum(_mm(em, we1_ref[...]) + be1_ref[...], 0.0),
             we2_ref[...]) + be2_ref[...]
    ze = jnp.zeros((_EMAX - _NE, _F), jnp.float32)
    qe = jnp.concatenate([et[:_NE], ze], axis=0)
    ce = jnp.concatenate([et[_NE:], ze], axis=0)
    Pe = _sinkhorn(_mmT(qe, ce) / _TEMP)
    s_edge = -jnp.sum(jnp.maximum(qe - _mm(Pe, ce), 0.0))

    score = s_edge * wv_ref[0, 0] + s_node * wv_ref[0, 1] + wv_ref[0, 2]
    out_ref[0] = jnp.full((1, 128), score, jnp.float32)


def kernel(node_features, edge_features, params, from_idx, to_idx):
    p = params
    nf3 = node_features.reshape(_B, 2 * _NN, _F)
    ef3 = edge_features.reshape(_B, 2 * _NE, 16)
    offs = jnp.repeat(jnp.arange(_B, dtype=jnp.int32) * (2 * _NN), 2 * _NE)
    fi3 = (from_idx - offs).reshape(_B, 1, 2 * _NE)
    ti3 = (to_idx - offs).reshape(_B, 1, 2 * _NE)

    Wr1 = p['Wr1']
    Wr1s = jnp.concatenate([Wr1[_F:2 * _F], Wr1[:_F], Wr1[2 * _F:]], axis=0)
    W1 = jnp.concatenate([p['Wm1'], Wr1s], axis=1)
    B1 = jnp.concatenate([p['bm1'], p['br1']])[None]
    wv = jnp.zeros((1, _F), jnp.float32)
    wv = wv.at[0, 0].set(p['Wagg'][0, 0]).at[0, 1].set(p['Wagg'][1, 0])
    wv = wv.at[0, 2].set(p['bagg'][0])

    weights = [
        p['W_enc_n'], p['b_enc_n'][None], p['W_enc_e'], p['b_enc_e'][None],
        W1, B1, p['Wm2'], p['bm2'][None], p['Wr2'], p['br2'][None],
        p['Wu1'], p['bu1'][None], p['Wu2'], p['bu2'][None],
        p['Wn1'], p['bn1'][None], p['Wn2'], p['bn2'][None],
        p['We1'], p['be1'][None], p['We2'], p['be2'][None], wv,
    ]

    def block3(a):
        return pl.BlockSpec((1,) + a.shape[1:], lambda i: (i, 0, 0))

    def whole(a):
        return pl.BlockSpec(a.shape, lambda i: (0,) * a.ndim)

    out = pl.pallas_call(
        _pair_body,
        grid=(_B,),
        in_specs=[block3(nf3), block3(ef3), block3(fi3), block3(ti3)]
                 + [whole(w) for w in weights],
        out_specs=pl.BlockSpec((1, 1, _F), lambda i: (i, 0, 0)),
        out_shape=jax.ShapeDtypeStruct((_B, 1, _F), jnp.float32),
        compiler_params=pltpu.CompilerParams(
            dimension_semantics=("parallel",)),
    )(nf3, ef3, fi3, ti3, *weights)
    return out[:, 0, 0]


# speedup vs baseline: 7.7206x; 5.3828x over previous
import jax
import jax.numpy as jnp
from jax.experimental import pallas as pl
from jax.experimental.pallas import tpu as pltpu

_B = 64
_PPB = 8
_NN = 30
_NP = 64
_NE = 64
_NMAX = 32
_EMAX = 80
_F = 128
_LAYERS = 3
_SITER = 20
_TEMP = 0.1


def _mm(a, b):
    return jax.lax.dot_general(a, b, (((1,), (0,)), ((), ())),
                               preferred_element_type=jnp.float32)


def _bmm(a, b):
    return jax.lax.dot_general(a, b, (((2,), (1,)), ((0,), (0,))),
                               preferred_element_type=jnp.float32)


def _bmmT(a, b):
    return jax.lax.dot_general(a, b, (((2,), (2,)), ((0,), (0,))),
                               preferred_element_type=jnp.float32)


def _bgather(oT, x):
    return jax.lax.dot_general(oT, x, (((1,), (1,)), ((0,), (0,))),
                               preferred_element_type=jnp.float32)


def _sinkhorn(la):
    for _ in range(_SITER):
        m = jnp.max(la, axis=2, keepdims=True)
        la = la - (m + jnp.log(jnp.sum(jnp.exp(la - m), axis=2, keepdims=True)))
        m = jnp.max(la, axis=1, keepdims=True)
        la = la - (m + jnp.log(jnp.sum(jnp.exp(la - m), axis=1, keepdims=True)))
    return jnp.exp(la)


def _block_body(nf_ref, ef_ref, fi_ref, ti_ref,
                wen_ref, ben_ref, wee_ref, bee_ref,
                w1_ref, b1_ref, wm2_ref, bm2_ref, wr2_ref, br2_ref,
                wu1_ref, bu1_ref, wu2_ref, bu2_ref,
                wn1_ref, bn1_ref, wn2_ref, bn2_ref,
                we1_ref, be1_ref, we2_ref, be2_ref, wv_ref,
                out_ref):
    P = _PPB
    NE2 = 2 * _NE
    nf = nf_ref[0]
    ef = ef_ref[0]
    W1 = w1_ref[...]; B1 = b1_ref[...]
    WM2 = wm2_ref[...]; BM2 = bm2_ref[...]
    WR2 = wr2_ref[...]; BR2 = br2_ref[...]
    WU1 = wu1_ref[...]; BU1 = bu1_ref[...]
    WU2 = wu2_ref[...]; BU2 = bu2_ref[...]

    h = _mm(nf, wen_ref[...]) + ben_ref[...]
    e = _mm(ef, wee_ref[...]) + bee_ref[...]

    niota = jax.lax.broadcasted_iota(jnp.int32, (P, _NP, NE2), 1)
    fi = fi_ref[0]
    ti = ti_ref[0]
    ofT = (niota == fi[:, None, :]).astype(jnp.float32)
    otT = (niota == ti[:, None, :]).astype(jnp.float32)

    def messages(hcur):
        h3 = hcur.reshape(P, _NP, _F)
        hf = _bgather(ofT, h3).reshape(P * NE2, _F)
        ht = _bgather(otT, h3).reshape(P * NE2, _F)
        x = jnp.concatenate([hf, ht, e], axis=1)
        y = jnp.maximum(_mm(x, W1) + B1, 0.0)
        m = _mm(y[:, :2 * _F], WM2) + BM2
        rm = _mm(y[:, 2 * _F:], WR2) + BR2
        return m, rm

    for _ in range(_LAYERS):
        m, rm = messages(h)
        agg = _bmm(otT, m.reshape(P, NE2, 2 * _F)) \
            + _bmm(ofT, rm.reshape(P, NE2, 2 * _F))
        u = jnp.concatenate([h, agg.reshape(P * _NP, 2 * _F)], axis=1)
        h = _mm(jnp.maximum(_mm(u, WU1) + BU1, 0.0), WU2) + BU2

    nt = _mm(jnp.maximum(_mm(h, wn1_ref[...]) + bn1_ref[...], 0.0),
             wn2_ref[...]) + bn2_ref[...]
    nt3 = nt.reshape(P, _NP, _F)
    zn = jnp.zeros((P, _NMAX - _NN, _F), jnp.float32)
    qn = jnp.concatenate([nt3[:, :_NN], zn], axis=1)
    cn = jnp.concatenate([nt3[:, _NN:2 * _NN], zn], axis=1)
    Pn = _sinkhorn(_bmmT(qn, cn) / _TEMP)
    s_node = jnp.sum(jnp.sum(jnp.maximum(qn - _bmm(Pn, cn), 0.0), axis=1),
                     axis=1, keepdims=True)

    m, rm = messages(h)
    em = m + rm
    et = _mm(jnp.maximum(_mm(em, we1_ref[...]) + be1_ref[...], 0.0),
             we2_ref[...]) + be2_ref[...]
    et3 = et.reshape(P, NE2, _F)
    ze = jnp.zeros((P, _EMAX - _NE, _F), jnp.float32)
    qe = jnp.concatenate([et3[:, :_NE], ze], axis=1)
    ce = jnp.concatenate([et3[:, _NE:], ze], axis=1)
    Pe = _sinkhorn(_bmmT(qe, ce) / _TEMP)
    s_edge = jnp.sum(jnp.sum(jnp.maximum(qe - _bmm(Pe, ce), 0.0), axis=1),
                     axis=1, keepdims=True)

    score = -s_edge * wv_ref[0, 0] - s_node * wv_ref[0, 1] + wv_ref[0, 2]
    out_ref[0] = jnp.broadcast_to(score, (P, _F))


def kernel(node_features, edge_features, params, from_idx, to_idx):
    p = params
    NB = _B // _PPB
    nf3 = node_features.reshape(_B, 2 * _NN, _F)
    nf3 = jnp.pad(nf3, ((0, 0), (0, _NP - 2 * _NN), (0, 0)))
    nf3 = nf3.reshape(NB, _PPB * _NP, _F)
    ef3 = edge_features.reshape(NB, _PPB * 2 * _NE, 16)
    offs = jnp.repeat(jnp.arange(_B, dtype=jnp.int32) * (2 * _NN), 2 * _NE)
    fi3 = (from_idx - offs).reshape(NB, _PPB, 2 * _NE)
    ti3 = (to_idx - offs).reshape(NB, _PPB, 2 * _NE)

    Wr1 = p['Wr1']
    Wr1s = jnp.concatenate([Wr1[_F:2 * _F], Wr1[:_F], Wr1[2 * _F:]], axis=0)
    W1 = jnp.concatenate([p['Wm1'], Wr1s], axis=1)
    B1 = jnp.concatenate([p['bm1'], p['br1']])[None]
    wv = jnp.zeros((1, _F), jnp.float32)
    wv = wv.at[0, 0].set(p['Wagg'][0, 0]).at[0, 1].set(p['Wagg'][1, 0])
    wv = wv.at[0, 2].set(p['bagg'][0])

    weights = [
        p['W_enc_n'], p['b_enc_n'][None], p['W_enc_e'], p['b_enc_e'][None],
        W1, B1, p['Wm2'], p['bm2'][None], p['Wr2'], p['br2'][None],
        p['Wu1'], p['bu1'][None], p['Wu2'], p['bu2'][None],
        p['Wn1'], p['bn1'][None], p['Wn2'], p['bn2'][None],
        p['We1'], p['be1'][None], p['We2'], p['be2'][None], wv,
    ]

    def block3(a):
        return pl.BlockSpec((1,) + a.shape[1:], lambda i: (i, 0, 0))

    def whole(a):
        return pl.BlockSpec(a.shape, lambda i: (0,) * a.ndim)

    out = pl.pallas_call(
        _block_body,
        grid=(NB,),
        in_specs=[block3(nf3), block3(ef3), block3(fi3), block3(ti3)]
                 + [whole(w) for w in weights],
        out_specs=pl.BlockSpec((1, _PPB, _F), lambda i: (i, 0, 0)),
        out_shape=jax.ShapeDtypeStruct((NB, _PPB, _F), jnp.float32),
        compiler_params=pltpu.CompilerParams(
            dimension_semantics=("parallel",)),
    )(nf3, ef3, fi3, ti3, *weights)
    return out[:, :, 0].reshape(_B)


# speedup vs baseline: 9.8089x; 1.2705x over previous
import jax
import jax.numpy as jnp
from jax.experimental import pallas as pl
from jax.experimental.pallas import tpu as pltpu

_B = 64
_PPB = 16
_NN = 30
_NP = 64
_NE = 64
_NMAX = 32
_EMAX = 80
_F = 128
_LAYERS = 3
_SITER = 20
_TEMP = 0.1


def _mm(a, b):
    return jax.lax.dot_general(a, b, (((1,), (0,)), ((), ())),
                               preferred_element_type=jnp.float32)


def _bmm(a, b):
    return jax.lax.dot_general(a, b, (((2,), (1,)), ((0,), (0,))),
                               preferred_element_type=jnp.float32)


def _bmmT(a, b):
    return jax.lax.dot_general(a, b, (((2,), (2,)), ((0,), (0,))),
                               preferred_element_type=jnp.float32)


def _bgather(oT, x):
    return jax.lax.dot_general(oT, x, (((1,), (1,)), ((0,), (0,))),
                               preferred_element_type=jnp.float32)


def _sinkhorn(la):
    for _ in range(_SITER):
        m = jnp.max(la, axis=2, keepdims=True)
        la = la - (m + jnp.log(jnp.sum(jnp.exp(la - m), axis=2, keepdims=True)))
        m = jnp.max(la, axis=1, keepdims=True)
        la = la - (m + jnp.log(jnp.sum(jnp.exp(la - m), axis=1, keepdims=True)))
    return jnp.exp(la)


def _block_body(nf_ref, ef_ref, fi_ref, ti_ref,
                wen_ref, ben_ref, wee_ref, bee_ref,
                w1_ref, b1_ref, wm2_ref, bm2_ref, wr2_ref, br2_ref,
                wu1_ref, bu1_ref, wu2_ref, bu2_ref,
                wn1_ref, bn1_ref, wn2_ref, bn2_ref,
                we1_ref, be1_ref, we2_ref, be2_ref, wv_ref,
                out_ref):
    P = _PPB
    NE2 = 2 * _NE
    nf = nf_ref[0]
    ef = ef_ref[0]
    W1 = w1_ref[...]; B1 = b1_ref[...]
    WM2 = wm2_ref[...]; BM2 = bm2_ref[...]
    WR2 = wr2_ref[...]; BR2 = br2_ref[...]
    WU1 = wu1_ref[...]; BU1 = bu1_ref[...]
    WU2 = wu2_ref[...]; BU2 = bu2_ref[...]

    h = _mm(nf, wen_ref[...]) + ben_ref[...]
    e = _mm(ef, wee_ref[...]) + bee_ref[...]

    niota = jax.lax.broadcasted_iota(jnp.int32, (P, _NP, NE2), 1)
    fi = fi_ref[0]
    ti = ti_ref[0]
    ofT = (niota == fi[:, None, :]).astype(jnp.float32)
    otT = (niota == ti[:, None, :]).astype(jnp.float32)

    def messages(hcur):
        h3 = hcur.reshape(P, _NP, _F)
        hf = _bgather(ofT, h3).reshape(P * NE2, _F)
        ht = _bgather(otT, h3).reshape(P * NE2, _F)
        x = jnp.concatenate([hf, ht, e], axis=1)
        y = jnp.maximum(_mm(x, W1) + B1, 0.0)
        m = _mm(y[:, :2 * _F], WM2) + BM2
        rm = _mm(y[:, 2 * _F:], WR2) + BR2
        return m, rm

    for _ in range(_LAYERS):
        m, rm = messages(h)
        agg = _bmm(otT, m.reshape(P, NE2, 2 * _F)) \
            + _bmm(ofT, rm.reshape(P, NE2, 2 * _F))
        u = jnp.concatenate([h, agg.reshape(P * _NP, 2 * _F)], axis=1)
        h = _mm(jnp.maximum(_mm(u, WU1) + BU1, 0.0), WU2) + BU2

    nt = _mm(jnp.maximum(_mm(h, wn1_ref[...]) + bn1_ref[...], 0.0),
             wn2_ref[...]) + bn2_ref[...]
    nt3 = nt.reshape(P, _NP, _F)
    zn = jnp.zeros((P, _NMAX - _NN, _F), jnp.float32)
    qn = jnp.concatenate([nt3[:, :_NN], zn], axis=1)
    cn = jnp.concatenate([nt3[:, _NN:2 * _NN], zn], axis=1)
    Pn = _sinkhorn(_bmmT(qn, cn) / _TEMP)
    s_node = jnp.sum(jnp.sum(jnp.maximum(qn - _bmm(Pn, cn), 0.0), axis=1),
                     axis=1, keepdims=True)

    m, rm = messages(h)
    em = m + rm
    et = _mm(jnp.maximum(_mm(em, we1_ref[...]) + be1_ref[...], 0.0),
             we2_ref[...]) + be2_ref[...]
    et3 = et.reshape(P, NE2, _F)
    ze = jnp.zeros((P, _EMAX - _NE, _F), jnp.float32)
    qe = jnp.concatenate([et3[:, :_NE], ze], axis=1)
    ce = jnp.concatenate([et3[:, _NE:], ze], axis=1)
    Pe = _sinkhorn(_bmmT(qe, ce) / _TEMP)
    s_edge = jnp.sum(jnp.sum(jnp.maximum(qe - _bmm(Pe, ce), 0.0), axis=1),
                     axis=1, keepdims=True)

    score = -s_edge * wv_ref[0, 0] - s_node * wv_ref[0, 1] + wv_ref[0, 2]
    out_ref[0] = jnp.broadcast_to(score, (P, _F))


def kernel(node_features, edge_features, params, from_idx, to_idx):
    p = params
    NB = _B // _PPB
    nf3 = node_features.reshape(_B, 2 * _NN, _F)
    nf3 = jnp.pad(nf3, ((0, 0), (0, _NP - 2 * _NN), (0, 0)))
    nf3 = nf3.reshape(NB, _PPB * _NP, _F)
    ef3 = edge_features.reshape(NB, _PPB * 2 * _NE, 16)
    offs = jnp.repeat(jnp.arange(_B, dtype=jnp.int32) * (2 * _NN), 2 * _NE)
    fi3 = (from_idx - offs).reshape(NB, _PPB, 2 * _NE)
    ti3 = (to_idx - offs).reshape(NB, _PPB, 2 * _NE)

    Wr1 = p['Wr1']
    Wr1s = jnp.concatenate([Wr1[_F:2 * _F], Wr1[:_F], Wr1[2 * _F:]], axis=0)
    W1 = jnp.concatenate([p['Wm1'], Wr1s], axis=1)
    B1 = jnp.concatenate([p['bm1'], p['br1']])[None]
    wv = jnp.zeros((1, _F), jnp.float32)
    wv = wv.at[0, 0].set(p['Wagg'][0, 0]).at[0, 1].set(p['Wagg'][1, 0])
    wv = wv.at[0, 2].set(p['bagg'][0])

    weights = [
        p['W_enc_n'], p['b_enc_n'][None], p['W_enc_e'], p['b_enc_e'][None],
        W1, B1, p['Wm2'], p['bm2'][None], p['Wr2'], p['br2'][None],
        p['Wu1'], p['bu1'][None], p['Wu2'], p['bu2'][None],
        p['Wn1'], p['bn1'][None], p['Wn2'], p['bn2'][None],
        p['We1'], p['be1'][None], p['We2'], p['be2'][None], wv,
    ]

    def block3(a):
        return pl.BlockSpec((1,) + a.shape[1:], lambda i: (i, 0, 0))

    def whole(a):
        return pl.BlockSpec(a.shape, lambda i: (0,) * a.ndim)

    out = pl.pallas_call(
        _block_body,
        grid=(NB,),
        in_specs=[block3(nf3), block3(ef3), block3(fi3), block3(ti3)]
                 + [whole(w) for w in weights],
        out_specs=pl.BlockSpec((1, _PPB, _F), lambda i: (i, 0, 0)),
        out_shape=jax.ShapeDtypeStruct((NB, _PPB, _F), jnp.float32),
        compiler_params=pltpu.CompilerParams(
            dimension_semantics=("parallel",)),
    )(nf3, ef3, fi3, ti3, *weights)
    return out[:, :, 0].reshape(_B)
